# baseline (speedup 1.0000x reference)
import functools

import jax
import jax.numpy as jnp
import numpy as np
from jax.experimental import pallas as pl
from jax.experimental.pallas import tpu as pltpu
from jax.experimental.pallas import tpu_sc as plsc

B = 16384
NUM_NUMERIC = 13
N_FIELDS = 26
CARD = 100000
EMB_DIM = 32
EMB_WIDTH = N_FIELDS * EMB_DIM
H1, H2 = 64, 32
EPS = 1e-5
NUM_IDX = B * N_FIELDS
GW = 128
BLK = 2048

CHUNK = 4096
NBLK = -(-CARD // CHUNK)
FG = 4
NG = -(-N_FIELDS // FG)
PLANE = NBLK * CHUNK
LIN_ROWS = NG * PLANE * FG


def _detranspose(tab_T):
    def body(in_ref, out_ref):
        x = in_ref[...]
        out_ref[0] = x.reshape(FG * EMB_DIM, CHUNK).T

    return pl.pallas_call(
        body,
        grid=(NG, NBLK),
        in_specs=[pl.BlockSpec((FG, EMB_DIM, CHUNK), lambda g, c: (g, 0, c))],
        out_specs=pl.BlockSpec((1, CHUNK, FG * EMB_DIM), lambda g, c: (g, c, 0)),
        out_shape=jax.ShapeDtypeStruct((NG, PLANE, FG * EMB_DIM), jnp.float32),
        compiler_params=pltpu.CompilerParams(
            dimension_semantics=("parallel", "parallel")
        ),
    )(tab_T)


def _sc_gather(tables_lin, flat_idx):
    mesh = plsc.VectorSubcoreMesh(core_axis_name="core", subcore_axis_name="subcore")

    @functools.partial(
        pl.kernel,
        out_type=jax.ShapeDtypeStruct((NUM_IDX, EMB_DIM), jnp.float32),
        mesh=mesh,
        compiler_params=pltpu.CompilerParams(use_tc_tiling_on_sc=False),
    )
    def gather_kernel(tab_hbm, idx_hbm, out_hbm):
        def body(idx_v, out_v):
            pltpu.sync_copy(tab_hbm.at[idx_v.at[0]], out_v)

        pltpu.emit_pipeline(
            body,
            grid=(NUM_IDX // GW,),
            in_specs=[pl.BlockSpec((1, GW), index_map=lambda i: (0, i))],
            out_specs=[pl.BlockSpec((GW, EMB_DIM), index_map=lambda i: (i, 0))],
            core_axis_name=("core", "subcore"),
            dimension_semantics=(pltpu.PARALLEL,),
        )(idx_hbm, out_hbm)

    return gather_kernel(tables_lin, flat_idx)


def _mlp_body(xn, em, w1n, w1e, b1r, g1r, be1r, w2, b2r, g2r, be2r, w3, b3r, out):
    s = np.float32(1.0 / np.sqrt(1.0 + EPS))
    h = jnp.dot(em[...], w1e[...], preferred_element_type=jnp.float32)
    h = h + jnp.dot(xn[...], w1n[...], preferred_element_type=jnp.float32)
    a1 = g1r[...] * s
    h = h * a1 + (b1r[...] * a1 + be1r[...])
    h = jnp.maximum(h, 0.0)
    h2 = jnp.dot(h, w2[...], preferred_element_type=jnp.float32)
    a2 = g2r[...] * s
    h2 = h2 * a2 + (b2r[...] * a2 + be2r[...])
    h2 = jnp.maximum(h2, 0.0)
    out[...] = jnp.dot(h2, w3[...], preferred_element_type=jnp.float32) + b3r[...]


def _mlp(x_num, emb, W1n, W1e, b1, g1, be1, W2, b2, g2, be2, W3, b3):
    grid = (B // BLK,)
    row_spec = lambda w: pl.BlockSpec((BLK, w), lambda i: (i, 0))
    full_spec = lambda a: pl.BlockSpec(a.shape, lambda i: (0, 0))
    args = (x_num, emb, W1n, W1e, b1, g1, be1, W2, b2, g2, be2, W3, b3)
    in_specs = [row_spec(NUM_NUMERIC), row_spec(EMB_WIDTH)] + [full_spec(a) for a in args[2:]]
    return pl.pallas_call(
        _mlp_body,
        grid=grid,
        in_specs=in_specs,
        out_specs=pl.BlockSpec((BLK, 1), lambda i: (i, 0)),
        out_shape=jax.ShapeDtypeStruct((B, 1), jnp.float32),
        compiler_params=pltpu.CompilerParams(dimension_semantics=("parallel",)),
    )(*args)


def kernel(x_num, x_cat, emb_tables, W1, b1, g1, be1, W2, b2, g2, be2, W3, b3):
    lin = _detranspose(jnp.swapaxes(emb_tables, 1, 2))
    tables_lin = lin.reshape(LIN_ROWS, EMB_DIM)
    f = jnp.arange(N_FIELDS, dtype=jnp.int32)
    offs = (4 * PLANE * (f // FG) + f % FG)[None, :]
    flat_idx = (x_cat * 4 + offs).reshape(1, NUM_IDX)
    emb = _sc_gather(tables_lin, flat_idx).reshape(B, EMB_WIDTH)
    W1n = W1[:NUM_NUMERIC]
    W1e = W1[NUM_NUMERIC:]
    vec = lambda v: v.reshape(1, -1)
    return _mlp(x_num, emb, W1n, W1e, vec(b1), vec(g1), vec(be1),
                W2, vec(b2), vec(g2), vec(be2), W3, vec(b3))


# speedup vs baseline: 1.1295x; 1.1295x over previous
import functools

import jax
import jax.numpy as jnp
import numpy as np
from jax.experimental import pallas as pl
from jax.experimental.pallas import tpu as pltpu
from jax.experimental.pallas import tpu_sc as plsc

B = 16384
NUM_NUMERIC = 13
N_FIELDS = 26
CARD = 100000
EMB_DIM = 32
EMB_WIDTH = N_FIELDS * EMB_DIM
H1, H2 = 64, 32
EPS = 1e-5
GW = 128
BLK = 2048

CHUNK = 4096
NBLK = -(-CARD // CHUNK)
FG = 4
NG = -(-N_FIELDS // FG)
PLANE = NBLK * CHUNK
NG_A = 4
NF_A = NG_A * FG
NF_B = N_FIELDS - NF_A
NG_B = NG - NG_A


def _detranspose(tab_T, g0, ng):
    def body(in_ref, out_ref):
        x = in_ref[...]
        out_ref[0] = x.reshape(FG * EMB_DIM, CHUNK).T

    return pl.pallas_call(
        body,
        grid=(ng, NBLK),
        in_specs=[pl.BlockSpec((FG, EMB_DIM, CHUNK), lambda g, c: (g0 + g, 0, c))],
        out_specs=pl.BlockSpec((1, CHUNK, FG * EMB_DIM), lambda g, c: (g, c, 0)),
        out_shape=jax.ShapeDtypeStruct((ng, PLANE, FG * EMB_DIM), jnp.float32),
        compiler_params=pltpu.CompilerParams(
            dimension_semantics=("parallel", "parallel")
        ),
    )(tab_T)


def _sc_gather(tables_lin, flat_idx, nidx):
    mesh = plsc.VectorSubcoreMesh(core_axis_name="core", subcore_axis_name="subcore")

    @functools.partial(
        pl.kernel,
        out_type=jax.ShapeDtypeStruct((nidx, EMB_DIM), jnp.float32),
        mesh=mesh,
        compiler_params=pltpu.CompilerParams(use_tc_tiling_on_sc=False),
    )
    def gather_kernel(tab_hbm, idx_hbm, out_hbm):
        def body(idx_v, out_v):
            pltpu.sync_copy(tab_hbm.at[idx_v.at[0]], out_v)

        pltpu.emit_pipeline(
            body,
            grid=(nidx // GW,),
            in_specs=[pl.BlockSpec((1, GW), index_map=lambda i: (0, i))],
            out_specs=[pl.BlockSpec((GW, EMB_DIM), index_map=lambda i: (i, 0))],
            core_axis_name=("core", "subcore"),
            dimension_semantics=(pltpu.PARALLEL,),
        )(idx_hbm, out_hbm)

    return gather_kernel(tables_lin, flat_idx)


def _mlp_body(xn, emA, emB, w1n, w1a, w1b, b1r, g1r, be1r, w2, b2r, g2r, be2r,
              w3, b3r, out):
    s = np.float32(1.0 / np.sqrt(1.0 + EPS))
    h = jnp.dot(emA[...], w1a[...], preferred_element_type=jnp.float32)
    h = h + jnp.dot(emB[...], w1b[...], preferred_element_type=jnp.float32)
    h = h + jnp.dot(xn[...], w1n[...], preferred_element_type=jnp.float32)
    a1 = g1r[...] * s
    h = h * a1 + (b1r[...] * a1 + be1r[...])
    h = jnp.maximum(h, 0.0)
    h2 = jnp.dot(h, w2[...], preferred_element_type=jnp.float32)
    a2 = g2r[...] * s
    h2 = h2 * a2 + (b2r[...] * a2 + be2r[...])
    h2 = jnp.maximum(h2, 0.0)
    out[...] = jnp.dot(h2, w3[...], preferred_element_type=jnp.float32) + b3r[...]


def _mlp(x_num, embA, embB, W1n, W1a, W1b, b1, g1, be1, W2, b2, g2, be2, W3, b3):
    grid = (B // BLK,)
    row_spec = lambda w: pl.BlockSpec((BLK, w), lambda i: (i, 0))
    full_spec = lambda a: pl.BlockSpec(a.shape, lambda i: (0, 0))
    args = (x_num, embA, embB, W1n, W1a, W1b, b1, g1, be1, W2, b2, g2, be2, W3, b3)
    in_specs = [
        row_spec(NUM_NUMERIC),
        row_spec(NF_A * EMB_DIM),
        row_spec(NF_B * EMB_DIM),
    ] + [full_spec(a) for a in args[3:]]
    return pl.pallas_call(
        _mlp_body,
        grid=grid,
        in_specs=in_specs,
        out_specs=pl.BlockSpec((BLK, 1), lambda i: (i, 0)),
        out_shape=jax.ShapeDtypeStruct((B, 1), jnp.float32),
        compiler_params=pltpu.CompilerParams(dimension_semantics=("parallel",)),
    )(*args)


def _half_idx(x_cat_half, nf):
    f = jnp.arange(nf, dtype=jnp.int32)
    offs = (4 * PLANE * (f // FG) + f % FG)[None, :]
    return (x_cat_half * 4 + offs).reshape(1, B * nf)


def kernel(x_num, x_cat, emb_tables, W1, b1, g1, be1, W2, b2, g2, be2, W3, b3):
    tab_T = jnp.swapaxes(emb_tables, 1, 2)
    linA = _detranspose(tab_T, 0, NG_A).reshape(NG_A * PLANE * FG, EMB_DIM)
    linB = _detranspose(tab_T, NG_A, NG_B).reshape(NG_B * PLANE * FG, EMB_DIM)
    idxA = _half_idx(x_cat[:, :NF_A], NF_A)
    idxB = _half_idx(x_cat[:, NF_A:], NF_B)
    embA = _sc_gather(linA, idxA, B * NF_A).reshape(B, NF_A * EMB_DIM)
    embB = _sc_gather(linB, idxB, B * NF_B).reshape(B, NF_B * EMB_DIM)
    W1n = W1[:NUM_NUMERIC]
    W1a = W1[NUM_NUMERIC:NUM_NUMERIC + NF_A * EMB_DIM]
    W1b = W1[NUM_NUMERIC + NF_A * EMB_DIM:]
    vec = lambda v: v.reshape(1, -1)
    return _mlp(x_num, embA, embB, W1n, W1a, W1b, vec(b1), vec(g1), vec(be1),
                W2, vec(b2), vec(g2), vec(be2), W3, vec(b3))
